# Initial kernel scaffold; baseline (speedup 1.0000x reference)
import jax
import jax.numpy as jnp
from jax.experimental import pallas as pl


def kernel(x, batch, W, b):
    raise NotImplementedError("write your pallas kernel here")


# speedup vs baseline: 3.5450x; 3.5450x over previous
import functools

import jax
import jax.numpy as jnp
from jax import lax
from jax.experimental import pallas as pl

N_NODES = 100000
IN_FEATS = 512
NUM_SEGMENTS = 1024

BLK = 2048
N_PAD = 102400
GRID = N_PAD // BLK


def _fused_body(x_ref, ids_ref, w_ref, b_ref, out_ref):
    i = pl.program_id(0)

    @pl.when(i == 0)
    def _init():
        out_ref[...] = jnp.zeros_like(out_ref)

    x = x_ref[...]
    t = jnp.dot(x, w_ref[...], preferred_element_type=jnp.float32)
    t = t + b_ref[0, 0]
    w = 1.0 / (1.0 + jnp.exp(-t))
    xw = x * w

    ids = ids_ref[0, 0, :]
    seg = lax.broadcasted_iota(jnp.int32, (NUM_SEGMENTS, BLK), 0)
    one_hot = (seg == ids[None, :]).astype(jnp.float32)
    out_ref[...] += jnp.dot(one_hot, xw, preferred_element_type=jnp.float32)


@jax.jit
def kernel(x, batch, W, b):
    pad = N_PAD - N_NODES
    xp = jnp.pad(x, ((0, pad), (0, 0)))
    bp = jnp.pad(batch, (0, pad)).reshape(GRID, 1, BLK)
    b2 = b.reshape(1, 1)

    out = pl.pallas_call(
        _fused_body,
        grid=(GRID,),
        in_specs=[
            pl.BlockSpec((BLK, IN_FEATS), lambda i: (i, 0)),
            pl.BlockSpec((1, 1, BLK), lambda i: (i, 0, 0)),
            pl.BlockSpec((IN_FEATS, 1), lambda i: (0, 0)),
            pl.BlockSpec((1, 1), lambda i: (0, 0)),
        ],
        out_specs=pl.BlockSpec((NUM_SEGMENTS, IN_FEATS), lambda i: (0, 0)),
        out_shape=jax.ShapeDtypeStruct((NUM_SEGMENTS, IN_FEATS), jnp.float32),
    )(xp, bp, W, b2)
    return out
